# Initial kernel scaffold; baseline (speedup 1.0000x reference)
import jax
import jax.numpy as jnp
from jax.experimental import pallas as pl


def kernel(patches, forward_indexes, backward_indexes):
    raise NotImplementedError("write your pallas kernel here")


# speedup vs baseline: 1.6013x; 1.6013x over previous
import functools

import jax
import jax.numpy as jnp
from jax import lax
from jax.experimental import pallas as pl
from jax.experimental.pallas import tpu as pltpu
from jax.experimental.pallas import tpu_sc as plsc

T = 1728
B = 32
C = 768
NMASK = 432
KEPT = T - NMASK
NROWS = KEPT * B
NW = 32
RPW = NROWS // NW
CH = 48
NCHUNK = RPW // CH
L = 16

_mesh = plsc.VectorSubcoreMesh(core_axis_name="c", subcore_axis_name="s")


@functools.partial(
    pl.kernel,
    mesh=_mesh,
    out_type=jax.ShapeDtypeStruct((NROWS, C), jnp.float32),
    scratch_types=[
        pltpu.VMEM((CH,), jnp.int32),
        pltpu.VMEM((CH, C), jnp.float32),
        pltpu.SemaphoreType.DMA,
    ],
)
def _gather_rows(patches_hbm, fwd_hbm, out_hbm, idx_v, rows_v, sem):
    wid = lax.axis_index("s") * 2 + lax.axis_index("c")
    base = wid * RPW

    def chunk(c, carry):
        jb = base + c * CH
        pltpu.sync_copy(fwd_hbm.at[pl.ds(jb, CH)], idx_v)
        for k in range(CH // L):
            lane_j = jb + k * L + lax.iota(jnp.int32, L)
            idx_v[pl.ds(k * L, L)] = (
                idx_v[pl.ds(k * L, L)] * B + lax.rem(lane_j, B)
            )
        pltpu.async_copy(patches_hbm.at[idx_v], rows_v, sem).wait()
        pltpu.sync_copy(rows_v, out_hbm.at[pl.ds(jb, CH)])
        return carry

    lax.fori_loop(0, NCHUNK, chunk, 0)


def kernel(patches, forward_indexes, backward_indexes):
    patches_2d = patches.reshape(T * B, C)
    fwd_flat = forward_indexes[:KEPT].astype(jnp.int32).reshape(-1)
    out_2d = _gather_rows(patches_2d, fwd_flat)
    kept = out_2d.reshape(KEPT, B, C)
    return (kept, forward_indexes, backward_indexes)


# speedup vs baseline: 1.9547x; 1.2207x over previous
import functools

import jax
import jax.numpy as jnp
from jax import lax
from jax.experimental import pallas as pl
from jax.experimental.pallas import tpu as pltpu
from jax.experimental.pallas import tpu_sc as plsc

T = 1728
B = 32
C = 768
NMASK = 432
KEPT = T - NMASK
NROWS = KEPT * B
NW = 32
RPW = NROWS // NW
CH = 72
NCHUNK = RPW // CH
L = 16

_mesh = plsc.VectorSubcoreMesh(core_axis_name="c", subcore_axis_name="s")


@functools.partial(
    pl.kernel,
    mesh=_mesh,
    out_type=jax.ShapeDtypeStruct((NROWS, C), jnp.float32),
    scratch_types=[
        pltpu.VMEM((RPW,), jnp.int32),
        pltpu.VMEM((2, CH, C), jnp.float32),
        pltpu.SemaphoreType.DMA,
        pltpu.SemaphoreType.DMA,
        pltpu.SemaphoreType.DMA,
        pltpu.SemaphoreType.DMA,
    ],
)
def _gather_rows(patches_hbm, fwd_hbm, out_hbm, idx_v, rows_v,
                 gsem0, gsem1, wsem0, wsem1):
    wid = lax.axis_index("s") * 2 + lax.axis_index("c")
    base = wid * RPW
    gsems = (gsem0, gsem1)
    wsems = (wsem0, wsem1)

    pltpu.sync_copy(fwd_hbm.at[pl.ds(base, RPW)], idx_v)
    for k in range(RPW // L):
        lane_j = base + k * L + lax.iota(jnp.int32, L)
        idx_v[pl.ds(k * L, L)] = idx_v[pl.ds(k * L, L)] * B + lax.rem(lane_j, B)

    gathers = [None] * NCHUNK
    writes = [None] * NCHUNK
    for c in range(NCHUNK):
        b = c % 2
        if c >= 2:
            writes[c - 2].wait()
        gathers[c] = pltpu.async_copy(
            patches_hbm.at[idx_v.at[pl.ds(c * CH, CH)]], rows_v.at[b], gsems[b])
        if c >= 1:
            p = c - 1
            gathers[p].wait()
            writes[p] = pltpu.async_copy(
                rows_v.at[p % 2], out_hbm.at[pl.ds(base + p * CH, CH)],
                wsems[p % 2])
    last = NCHUNK - 1
    gathers[last].wait()
    writes[last] = pltpu.async_copy(
        rows_v.at[last % 2], out_hbm.at[pl.ds(base + last * CH, CH)],
        wsems[last % 2])
    writes[last - 1].wait()
    writes[last].wait()


def kernel(patches, forward_indexes, backward_indexes):
    patches_2d = patches.reshape(T * B, C)
    fwd_flat = forward_indexes[:KEPT].astype(jnp.int32).reshape(-1)
    out_2d = _gather_rows(patches_2d, fwd_flat)
    kept = out_2d.reshape(KEPT, B, C)
    return (kept, forward_indexes, backward_indexes)
